# Initial kernel scaffold; baseline (speedup 1.0000x reference)
import jax
import jax.numpy as jnp
from jax.experimental import pallas as pl


def kernel(x, pos, z_time, flag, batch_idx, ptr, params):
    raise NotImplementedError("write your pallas kernel here")


# speedup vs baseline: 1.5088x; 1.5088x over previous
import jax
import jax.numpy as jnp
import numpy as np
from jax.experimental import pallas as pl

B = 8
NPER = 1250
N = B * NPER
IN_CH = 128
HID = 64
SPACE = 4
PROP = 32
K = 16
HEADS = 4
GMT_K = 5
NCLS = 4
NSTATION = 40


def _seg_sum(d, ids, n):
    return jax.ops.segment_sum(d, ids, num_segments=n)


def _seg_max(d, ids, n):
    return jax.ops.segment_max(d, ids, num_segments=n)


def _station_spread(xy, z_q, ori, batch_idx):
    gid = batch_idx * (NSTATION * 2) + z_q * 2 + ori
    G = B * NSTATION * 2
    ones = jnp.ones((N,), jnp.float32)
    cnt = _seg_sum(ones, gid, G)
    safe = jnp.maximum(cnt, 1.0)
    cmean = _seg_sum(xy, gid, G) / safe
    dist = jnp.abs(xy - cmean[gid])
    sqm = _seg_sum(xy * xy, gid, G) / safe
    var = jnp.clip(sqm - cmean * cmean, 0.0, None)
    std = jnp.sqrt(var)[gid]
    nev = _seg_sum(ones, batch_idx, B)
    frac = cnt[gid] / jnp.maximum(nev[batch_idx], 1.0)
    return jnp.stack([dist, std, frac], axis=-1)


def _knn_graph(s):
    sb = s.reshape(B, NPER, -1)

    def per(se):
        d2 = jnp.sum((se[:, None, :] - se[None, :, :]) ** 2, axis=-1)
        d2 = d2 + jnp.eye(NPER, dtype=d2.dtype) * 1e12
        _, idx = jax.lax.top_k(-d2, K)
        return idx

    idx = jax.vmap(per)(sb)
    return idx


def _gravnet(x, z, p):
    s = x @ p['Ws'] + p['bs']
    h = x @ p['Wh'] + p['bh']
    idx = _knn_graph(s)
    sb = s.reshape(B, NPER, SPACE)
    hb = h.reshape(B, NPER, PROP)
    zb = z.reshape(B, NPER)
    s_src = jax.vmap(lambda se, ie: se[ie])(sb, idx)
    h_src = jax.vmap(lambda he, ie: he[ie])(hb, idx)
    z_src = jax.vmap(lambda ze, ie: ze[ie])(zb, idx)
    ew = jnp.exp(-10.0 * jnp.sum((s_src - sb[:, :, None, :]) ** 2, axis=-1))
    m = h_src * ew[..., None]
    agg_mean = jnp.sum(m, axis=2) / K
    agg_max = jnp.max(m, axis=2)
    dz = z_src - zb[:, :, None]
    das = jnp.sum(jnp.abs(dz), axis=2)
    F = dz / (das[..., None] + 1e-08)
    agg_smooth = jnp.sum(m * jnp.abs(F)[..., None], axis=2) / K
    agg_deriv = jnp.sum(m * F[..., None], axis=2)
    agg_all = jnp.concatenate([agg_mean, agg_max, agg_smooth, agg_deriv], axis=-1)
    agg_all = agg_all.reshape(N, 4 * PROP)
    out = x @ p['Wskip'] + agg_all @ p['Wagg'] + p['bagg']
    mu = jnp.mean(out, axis=0)
    var = jnp.var(out, axis=0)
    out = p['gamma'] * (out - mu) / jnp.sqrt(var + 1e-05) + p['beta']
    return jax.nn.relu(out)


def _mha(Q, Kx, p):
    b, m, d = Q.shape
    n = Kx.shape[1]
    dh = d // HEADS
    q = (Q @ p['Wq']).reshape(b, m, HEADS, dh).transpose(0, 2, 1, 3)
    k = (Kx @ p['Wk']).reshape(b, n, HEADS, dh).transpose(0, 2, 1, 3)
    v = (Kx @ p['Wv']).reshape(b, n, HEADS, dh).transpose(0, 2, 1, 3)
    att = jax.nn.softmax(jnp.einsum('bhmd,bhnd->bhmn', q, k) / np.sqrt(float(dh)), axis=-1)
    o = jnp.einsum('bhmn,bhnd->bhmd', att, v).transpose(0, 2, 1, 3).reshape(b, m, d)
    return o @ p['Wo']


def _mab(Q, Kx, p):
    H = Q + _mha(Q, Kx, p)
    return H + jax.nn.relu(H @ p['Wff'] + p['bff'])


def _gmt(xn, params):
    xd = xn.reshape(B, NPER, HID)
    S1 = jnp.broadcast_to(params['S1'], (B, GMT_K, HID))
    hq = _mab(S1, xd, params['pma1'])
    hq = _mab(hq, hq, params['sab'])
    S2 = jnp.broadcast_to(params['S2'], (B, 1, HID))
    out = _mab(S2, hq, params['pma2'])
    return out[:, 0, :]


def _identity_pallas(x):
    def body(x_ref, o_ref):
        o_ref[...] = x_ref[...]

    return pl.pallas_call(
        body, out_shape=jax.ShapeDtypeStruct(x.shape, x.dtype))(x)


def kernel(x, pos, z_time, flag, batch_idx, ptr, params):
    z = z_time[:, 0]
    xy = pos[:, 0]
    z_q = jnp.clip(z.astype(jnp.int32), 0, NSTATION - 1)
    feats = _station_spread(xy, z_q, flag.astype(jnp.int32), batch_idx.astype(jnp.int32))
    h = jnp.concatenate([x, feats], axis=-1)
    for p in params['grav']:
        h = _gravnet(h, z, p)
    g = _gmt(h, params)
    g = jax.nn.relu(g @ params['Wc1'] + params['bc1'])
    g = jax.nn.relu(g @ params['Wc2'] + params['bc2'])
    out = g @ params['Wc3'] + params['bc3']
    return _identity_pallas(out)


# speedup vs baseline: 12.3491x; 8.1848x over previous
import functools

import jax
import jax.numpy as jnp
import numpy as np
from jax.experimental import pallas as pl
from jax.experimental.pallas import tpu as pltpu

B = 8
NPER = 1250
NPAD = 1280
N = B * NPER
IN_CH = 128
HID = 64
SPACE = 4
PROP = 32
K = 16
HEADS = 4
GMT_K = 5
NCLS = 4
NSTATION = 40
NGRP = NSTATION * 2


def _station_body(xy_ref, z_ref, fl_ref, o_ref):
    xy = xy_ref[0]
    z = z_ref[0]
    fl = fl_ref[0]
    z_q = jnp.clip(z.astype(jnp.int32), 0, NSTATION - 1)
    gid = z_q * 2 + fl
    giota = jax.lax.broadcasted_iota(jnp.int32, (NPAD, NGRP), 1)
    riota = jax.lax.broadcasted_iota(jnp.int32, (NPAD, 1), 0)
    valid = (riota < NPER).astype(jnp.float32)
    oh = jnp.where(gid == giota, 1.0, 0.0)
    ohm = oh * valid
    ones = jnp.ones((NPAD, 1), jnp.float32)
    contrib = jnp.concatenate([ones, xy, xy * xy], axis=1)
    g = jax.lax.dot_general(ohm, contrib, (((0,), (0,)), ((), ())),
                            preferred_element_type=jnp.float32,
                            precision=jax.lax.Precision.HIGHEST)
    cnt = g[:, 0:1]
    safe = jnp.maximum(cnt, 1.0)
    mean = g[:, 1:2] / safe
    sqm = g[:, 2:3] / safe
    var = jnp.clip(sqm - mean * mean, 0.0, None)
    std = jnp.sqrt(var)
    stat = jnp.concatenate([mean, std, cnt], axis=1)
    back = jnp.dot(oh, stat, preferred_element_type=jnp.float32,
                   precision=jax.lax.Precision.HIGHEST)
    dist = jnp.abs(xy - back[:, 0:1])
    frac = back[:, 2:3] * (1.0 / NPER)
    o_ref[0] = jnp.concatenate(
        [dist, back[:, 1:2], frac, jnp.zeros((NPAD, 1), jnp.float32)], axis=1)


def _station(xy8, z8, fl8):
    return pl.pallas_call(
        _station_body,
        grid=(B,),
        in_specs=[
            pl.BlockSpec((1, NPAD, 1), lambda i: (i, 0, 0)),
            pl.BlockSpec((1, NPAD, 1), lambda i: (i, 0, 0)),
            pl.BlockSpec((1, NPAD, 1), lambda i: (i, 0, 0)),
        ],
        out_specs=pl.BlockSpec((1, NPAD, 4), lambda i: (i, 0, 0)),
        out_shape=jax.ShapeDtypeStruct((B, NPAD, 4), jnp.float32),
    )(xy8, z8, fl8)


RB = 256


def _gravnet_body(h_ref, hr_ref, z_ref, zr_ref, ws_ref, bs_ref, wh_ref,
                  bh_ref, wskip_ref, wagg_ref, bagg_ref, o_ref):
    j = pl.program_id(1)
    h_e = h_ref[0]
    h_r = hr_ref[0]
    z_e = z_ref[0]
    z_r = zr_ref[0]
    s_e = jnp.dot(h_e, ws_ref[...], preferred_element_type=jnp.float32) + bs_ref[...]
    s_r = jnp.dot(h_r, ws_ref[...], preferred_element_type=jnp.float32) + bs_ref[...]
    hp_e = jnp.dot(h_e, wh_ref[...], preferred_element_type=jnp.float32) + bh_ref[...]
    skip_r = jnp.dot(h_r, wskip_ref[...], preferred_element_type=jnp.float32)

    tbl = jnp.concatenate(
        [hp_e, z_e, jnp.zeros((NPAD, 31), jnp.float32)], axis=1)

    s_t = s_e.T
    d2 = jnp.zeros((RB, NPAD), jnp.float32)
    for d in range(SPACE):
        diff = s_r[:, d:d + 1] - s_t[d:d + 1, :]
        d2 = d2 + diff * diff
    riota = jax.lax.broadcasted_iota(jnp.int32, (RB, NPAD), 0) + j * RB
    ciota = jax.lax.broadcasted_iota(jnp.int32, (RB, NPAD), 1)
    d2 = d2 + jnp.where(riota == ciota, 1e12, 0.0)
    d2 = jnp.where(ciota >= NPER, 1e30, d2)

    sum_m = jnp.zeros((RB, PROP), jnp.float32)
    max_m = jnp.full((RB, PROP), -1e30, jnp.float32)
    das = jnp.zeros((RB, 1), jnp.float32)
    sum_absdz = jnp.zeros((RB, PROP), jnp.float32)
    sum_dz = jnp.zeros((RB, PROP), jnp.float32)

    for _ in range(K):
        minval = jnp.min(d2, axis=1, keepdims=True)
        eq = d2 == minval
        idxsel = jnp.min(jnp.where(eq, ciota, NPAD + 1), axis=1,
                         keepdims=True)
        sel = ciota == idxsel
        d2 = jnp.where(sel, 1e30, d2)
        onehot = jnp.where(sel, 1.0, 0.0)
        g = jnp.dot(onehot, tbl, preferred_element_type=jnp.float32,
                    precision=jax.lax.Precision.HIGHEST)
        h_sel = g[:, :PROP]
        z_sel = g[:, PROP:PROP + 1]
        ew = jnp.exp(-10.0 * minval)
        m = h_sel * ew
        sum_m = sum_m + m
        max_m = jnp.maximum(max_m, m)
        dz = z_sel - z_r
        adz = jnp.abs(dz)
        das = das + adz
        sum_absdz = sum_absdz + m * adz
        sum_dz = sum_dz + m * dz

    inv = 1.0 / (das + 1e-08)
    agg = jnp.concatenate(
        [sum_m * (1.0 / K), max_m, sum_absdz * inv * (1.0 / K), sum_dz * inv],
        axis=1)
    out = skip_r + jnp.dot(agg, wagg_ref[...],
                           preferred_element_type=jnp.float32) + bagg_ref[...]
    o_ref[0] = out


def _gravnet_layer(h8, z8, p, ch):
    return pl.pallas_call(
        _gravnet_body,
        grid=(B, NPAD // RB),
        in_specs=[
            pl.BlockSpec((1, NPAD, ch), lambda i, j: (i, 0, 0)),
            pl.BlockSpec((1, RB, ch), lambda i, j: (i, j, 0)),
            pl.BlockSpec((1, NPAD, 1), lambda i, j: (i, 0, 0)),
            pl.BlockSpec((1, RB, 1), lambda i, j: (i, j, 0)),
            pl.BlockSpec((ch, SPACE), lambda i, j: (0, 0)),
            pl.BlockSpec((SPACE,), lambda i, j: (0,)),
            pl.BlockSpec((ch, PROP), lambda i, j: (0, 0)),
            pl.BlockSpec((PROP,), lambda i, j: (0,)),
            pl.BlockSpec((ch, HID), lambda i, j: (0, 0)),
            pl.BlockSpec((4 * PROP, HID), lambda i, j: (0, 0)),
            pl.BlockSpec((HID,), lambda i, j: (0,)),
        ],
        out_specs=pl.BlockSpec((1, RB, HID), lambda i, j: (i, j, 0)),
        out_shape=jax.ShapeDtypeStruct((B, NPAD, HID), jnp.float32),
        compiler_params=pltpu.CompilerParams(
            dimension_semantics=("arbitrary", "arbitrary")),
    )(h8, h8, z8, z8, p['Ws'], p['bs'], p['Wh'], p['bh'], p['Wskip'],
      p['Wagg'], p['bagg'])


def _norm_body(x_ref, g_ref, b_ref, o_ref):
    x = x_ref[...].reshape(B * NPAD, HID)
    riota = jax.lax.broadcasted_iota(jnp.int32, (B * NPAD, 1), 0)
    valid = (jax.lax.rem(riota, NPAD) < NPER).astype(jnp.float32)
    xm = x * valid
    mu = jnp.sum(xm, axis=0, keepdims=True) * (1.0 / N)
    d = (x - mu) * valid
    var = jnp.sum(d * d, axis=0, keepdims=True) * (1.0 / N)
    y = g_ref[...] * (x - mu) / jnp.sqrt(var + 1e-05) + b_ref[...]
    y = jnp.maximum(y, 0.0)
    o_ref[...] = y.reshape(B, NPAD, HID)


def _norm(x8, gamma, beta):
    return pl.pallas_call(
        _norm_body,
        out_shape=jax.ShapeDtypeStruct((B, NPAD, HID), jnp.float32),
    )(x8, gamma, beta)


def _mha_p(Q, Kx, wq, wk, wv, wo, mask_cols):
    m = Q.shape[0]
    n = Kx.shape[0]
    dh = HID // HEADS
    q = jnp.dot(Q, wq, preferred_element_type=jnp.float32)
    k = jnp.dot(Kx, wk, preferred_element_type=jnp.float32)
    v = jnp.dot(Kx, wv, preferred_element_type=jnp.float32)
    outs = []
    scale = 1.0 / np.sqrt(float(dh))
    for hh in range(HEADS):
        qh = q[:, hh * dh:(hh + 1) * dh]
        kh = k[:, hh * dh:(hh + 1) * dh]
        vh = v[:, hh * dh:(hh + 1) * dh]
        logits = jax.lax.dot_general(
            qh, kh, (((1,), (1,)), ((), ())),
            preferred_element_type=jnp.float32) * scale
        if mask_cols is not None:
            logits = jnp.where(mask_cols, logits, -1e30)
        mx = jnp.max(logits, axis=1, keepdims=True)
        e = jnp.exp(logits - mx)
        p = e / jnp.sum(e, axis=1, keepdims=True)
        outs.append(jnp.dot(p, vh, preferred_element_type=jnp.float32))
    o = jnp.concatenate(outs, axis=1)
    return jnp.dot(o, wo, preferred_element_type=jnp.float32)


def _mab_p(Q, Kx, w, mask_cols):
    wq, wk, wv, wo, wff, bff = w
    H = Q + _mha_p(Q, Kx, wq, wk, wv, wo, mask_cols)
    return H + jnp.maximum(
        jnp.dot(H, wff, preferred_element_type=jnp.float32) + bff, 0.0)


def _gmt_body(h_ref, s1_ref, s2_ref,
              q1, k1, v1, o1, f1, fb1,
              q2, k2, v2, o2, f2, fb2,
              q3, k3, v3, o3, f3, fb3,
              wc1, bc1, wc2, bc2, wc3, bc3, o_ref):
    h_e = h_ref[0]
    ciota = jax.lax.broadcasted_iota(jnp.int32, (GMT_K, NPAD), 1)
    mask1 = ciota < NPER
    w1 = (q1[...], k1[...], v1[...], o1[...], f1[...], fb1[...])
    w2 = (q2[...], k2[...], v2[...], o2[...], f2[...], fb2[...])
    w3 = (q3[...], k3[...], v3[...], o3[...], f3[...], fb3[...])
    hq = _mab_p(s1_ref[...], h_e, w1, mask1)
    hq = _mab_p(hq, hq, w2, None)
    out = _mab_p(s2_ref[...], hq, w3, None)
    g = jnp.maximum(jnp.dot(out, wc1[...],
                            preferred_element_type=jnp.float32) + bc1[...], 0.0)
    g = jnp.maximum(jnp.dot(g, wc2[...],
                            preferred_element_type=jnp.float32) + bc2[...], 0.0)
    o_ref[0] = jnp.dot(g, wc3[...],
                       preferred_element_type=jnp.float32) + bc3[...]


def _gmt(h8, params):
    mw = lambda p: (p['Wq'], p['Wk'], p['Wv'], p['Wo'], p['Wff'], p['bff'])
    args = (h8, params['S1'], params['S2'],
            *mw(params['pma1']), *mw(params['sab']), *mw(params['pma2']),
            params['Wc1'], params['bc1'], params['Wc2'], params['bc2'],
            params['Wc3'], params['bc3'])
    wspec = lambda a: pl.BlockSpec(a.shape, lambda i: (0,) * a.ndim)
    in_specs = [pl.BlockSpec((1, NPAD, HID), lambda i: (i, 0, 0))]
    in_specs += [wspec(a) for a in args[1:]]
    return pl.pallas_call(
        _gmt_body,
        grid=(B,),
        in_specs=in_specs,
        out_specs=pl.BlockSpec((1, 1, NCLS), lambda i: (i, 0, 0)),
        out_shape=jax.ShapeDtypeStruct((B, 1, NCLS), jnp.float32),
    )(*args)


def _pad_events(a):
    a = a.reshape(B, NPER, -1)
    return jnp.pad(a, ((0, 0), (0, NPAD - NPER), (0, 0)))


def kernel(x, pos, z_time, flag, batch_idx, ptr, params):
    xy8 = _pad_events(pos[:, 0:1])
    z8 = _pad_events(z_time)
    fl8 = _pad_events(flag.astype(jnp.int32)[:, None])

    feats = _station(xy8, z8, fl8)[:, :, :3]
    x8 = _pad_events(x)
    ch1 = IN_CH + 3
    ch1p = 136
    h8 = jnp.concatenate(
        [x8, feats, jnp.zeros((B, NPAD, ch1p - ch1), jnp.float32)], axis=2)

    p0, p1 = params['grav']
    pad_w = lambda w: jnp.pad(w, ((0, ch1p - ch1), (0, 0)))
    p0p = dict(p0, Ws=pad_w(p0['Ws']), Wh=pad_w(p0['Wh']),
               Wskip=pad_w(p0['Wskip']))

    out1 = _gravnet_layer(h8, z8, p0p, ch1p)
    h1 = _norm(out1, p0['gamma'], p0['beta'])
    out2 = _gravnet_layer(h1, z8, p1, HID)
    h2 = _norm(out2, p1['gamma'], p1['beta'])

    return _gmt(h2, params)[:, 0, :]
